# baseline (speedup 1.0000x reference)
import functools
import math

import jax
import jax.numpy as jnp
from jax import lax
from jax.experimental import pallas as pl
from jax.experimental.pallas import tpu as pltpu
from jax.experimental.pallas import tpu_sc as plsc

_NUM_LEVELS = 16
_LEVEL_DIM = 2
_BASE_RES = 16
_MAX_RES = 2048
_LOG2_HASH = 19
_HASH = 2 ** _LOG2_HASH
_SCALE = math.exp((math.log(_MAX_RES) - math.log(_BASE_RES)) / (_NUM_LEVELS - 1))
_RES = [int(_BASE_RES * _SCALE ** l) for l in range(_NUM_LEVELS)]
_R2M = [(r * r) % _HASH for r in _RES]

_NC = 2
_NS = 16
_NW = _NC * _NS

_P = 1024
_W = _NUM_LEVELS * _LEVEL_DIM


def _sc_body(xf, tabw, resf_h, resi_h, r2m_h, out,
             cx_v, resf_v, resi_v, r2m_v, fidx_v, gat_v, sem,
             *, n_points):
    ppw = n_points // _NW
    nchunk = ppw // _P
    wid = lax.axis_index("s") * _NC + lax.axis_index("c")
    base = wid * ppw

    pltpu.sync_copy(resf_h, resf_v)
    pltpu.sync_copy(resi_h, resi_v)
    pltpu.sync_copy(r2m_h, r2m_v)

    for chunk in range(nchunk):
        cb = base + chunk * _P
        pltpu.sync_copy(xf.at[pl.ds(cb * 3, _P * 3)], cx_v)

        def grp_body(g, _):
            resf = resf_v[...]
            resi = resi_v[...]
            r2m = r2m_v[...]
            lane = lax.iota(jnp.int32, 16)
            lvl = lane << _LOG2_HASH
            half = lane >> 1
            parity = lane & 1
            a = [cx_v[pl.ds(g * 48 + 16 * k, 16)] for k in range(3)]
            for j in range(16):
                i0 = (resf * a[(3 * j) // 16][(3 * j) % 16]).astype(jnp.int32)
                i1 = (resf * a[(3 * j + 1) // 16][(3 * j + 1) % 16]
                      ).astype(jnp.int32)
                i2 = (resf * a[(3 * j + 2) // 16][(3 * j + 2) % 16]
                      ).astype(jnp.int32)
                i0 = jnp.where(i0 >= resi, i0 - resi, i0)
                i1 = jnp.where(i1 >= resi, i1 - resi, i1)
                i2 = jnp.where(i2 >= resi, i2 - resi, i2)
                h = i0 + i1 * resi + i2 * r2m
                fidx = (h & (_HASH - 1)) | lvl
                lo = (fidx.at[half].get(mode="promise_in_bounds") << 1) + parity
                hi = (fidx.at[half + 8].get(mode="promise_in_bounds") << 1
                      ) + parity
                p = g * 16 + j
                fidx_v[pl.ds(p * _W, 16)] = lo
                fidx_v[pl.ds(p * _W + 16, 16)] = hi
            return 0

        lax.fori_loop(0, _P // 16, grp_body, 0)

        pltpu.async_copy(tabw.at[fidx_v], gat_v, sem).wait()
        pltpu.sync_copy(gat_v, out.at[pl.ds(cb * _W, _P * _W)])


def kernel(x, tables):
    b0, b1, _ = x.shape
    n = b0 * b1
    xf = x.reshape(n * 3)
    tabw = tables.reshape(-1)
    resf = jnp.array(_RES, dtype=jnp.float32)
    resi = jnp.array(_RES, dtype=jnp.int32)
    r2m = jnp.array(_R2M, dtype=jnp.int32)

    mesh = plsc.VectorSubcoreMesh(core_axis_name="c", subcore_axis_name="s")
    run = pl.kernel(
        functools.partial(_sc_body, n_points=n),
        out_type=jax.ShapeDtypeStruct((n * _W,), jnp.float32),
        mesh=mesh,
        compiler_params=pltpu.CompilerParams(use_tc_tiling_on_sc=False),
        scratch_types=[
            pltpu.VMEM((_P * 3,), jnp.float32),
            pltpu.VMEM((_NUM_LEVELS,), jnp.float32),
            pltpu.VMEM((_NUM_LEVELS,), jnp.int32),
            pltpu.VMEM((_NUM_LEVELS,), jnp.int32),
            pltpu.VMEM((_P * _W,), jnp.int32),
            pltpu.VMEM((_P * _W,), jnp.float32),
            pltpu.SemaphoreType.DMA,
        ],
    )
    out = run(xf, tabw, resf, resi, r2m)
    return out.reshape(b0, b1, _W)


# speedup vs baseline: 16.4754x; 16.4754x over previous
import functools
import math

import jax
import jax.numpy as jnp
from jax import lax
from jax.experimental import pallas as pl
from jax.experimental.pallas import tpu as pltpu
from jax.experimental.pallas import tpu_sc as plsc

_NUM_LEVELS = 16
_LEVEL_DIM = 2
_BASE_RES = 16
_MAX_RES = 2048
_LOG2_HASH = 19
_HASH = 2 ** _LOG2_HASH
_SCALE = math.exp((math.log(_MAX_RES) - math.log(_BASE_RES)) / (_NUM_LEVELS - 1))
_RES = [int(_BASE_RES * _SCALE ** l) for l in range(_NUM_LEVELS)]
_R2M = [(r * r) % _HASH for r in _RES]

_NC = 2
_NS = 16
_NW = _NC * _NS

_P = 1024
_W = _NUM_LEVELS * _LEVEL_DIM


def _sc_body(xf, tabw, resf_h, resi_h, r2m_h, out,
             cx_v, resf_v, resi_v, r2m_v, fidx_v, gat_v, sem,
             *, n_points):
    ppw = n_points // _NW
    nchunk = ppw // _P
    wid = lax.axis_index("s") * _NC + lax.axis_index("c")
    base = wid * ppw

    pltpu.sync_copy(resf_h, resf_v)
    pltpu.sync_copy(resi_h, resi_v)
    pltpu.sync_copy(r2m_h, r2m_v)

    for chunk in range(nchunk):
        cb = base + chunk * _P
        pltpu.sync_copy(xf.at[pl.ds(cb * 3, _P * 3)], cx_v)

        def grp_body(g, _):
            resf = resf_v[...]
            resi = resi_v[...]
            r2m = r2m_v[...]
            lane = lax.iota(jnp.int32, 16)
            lvl = lane << _LOG2_HASH
            half = lane >> 1
            parity = lane & 1
            a = [cx_v[pl.ds(g * 48 + 16 * k, 16)] for k in range(3)]
            for j in range(16):
                i0 = (resf * a[(3 * j) // 16][(3 * j) % 16]).astype(jnp.int32)
                i1 = (resf * a[(3 * j + 1) // 16][(3 * j + 1) % 16]
                      ).astype(jnp.int32)
                i2 = (resf * a[(3 * j + 2) // 16][(3 * j + 2) % 16]
                      ).astype(jnp.int32)
                i0 = jnp.where(i0 >= resi, i0 - resi, i0)
                i1 = jnp.where(i1 >= resi, i1 - resi, i1)
                i2 = jnp.where(i2 >= resi, i2 - resi, i2)
                h = i0 + i1 * resi + i2 * r2m
                fidx = (h & (_HASH - 1)) | lvl
                gl = fidx.at[half].get(mode="promise_in_bounds")
                gh = fidx.at[half + 8].get(mode="promise_in_bounds")
                lo = ((gl >> 7) << 8) | (parity << 7) | (gl & 127)
                hi = ((gh >> 7) << 8) | (parity << 7) | (gh & 127)
                p = g * 16 + j
                fidx_v[pl.ds(p * _W, 16)] = lo
                fidx_v[pl.ds(p * _W + 16, 16)] = hi
            return 0

        lax.fori_loop(0, _P // 16, grp_body, 0)

        pltpu.async_copy(tabw.at[fidx_v], gat_v, sem).wait()
        pltpu.sync_copy(gat_v, out.at[pl.ds(cb * _W, _P * _W)])


def kernel(x, tables):
    b0, b1, _ = x.shape
    n = b0 * b1
    xf = x.reshape(n * 3)
    tabw = tables.reshape(_NUM_LEVELS, _HASH // 128, 128, _LEVEL_DIM)
    tabw = tabw.transpose(0, 1, 3, 2).reshape(-1)
    resf = jnp.array(_RES, dtype=jnp.float32)
    resi = jnp.array(_RES, dtype=jnp.int32)
    r2m = jnp.array(_R2M, dtype=jnp.int32)

    mesh = plsc.VectorSubcoreMesh(core_axis_name="c", subcore_axis_name="s")
    run = pl.kernel(
        functools.partial(_sc_body, n_points=n),
        out_type=jax.ShapeDtypeStruct((n * _W,), jnp.float32),
        mesh=mesh,
        compiler_params=pltpu.CompilerParams(use_tc_tiling_on_sc=False),
        scratch_types=[
            pltpu.VMEM((_P * 3,), jnp.float32),
            pltpu.VMEM((_NUM_LEVELS,), jnp.float32),
            pltpu.VMEM((_NUM_LEVELS,), jnp.int32),
            pltpu.VMEM((_NUM_LEVELS,), jnp.int32),
            pltpu.VMEM((_P * _W,), jnp.int32),
            pltpu.VMEM((_P * _W,), jnp.float32),
            pltpu.SemaphoreType.DMA,
        ],
    )
    out = run(xf, tabw, resf, resi, r2m)
    return out.reshape(b0, b1, _W)


# speedup vs baseline: 30.7954x; 1.8692x over previous
import functools
import math

import jax
import jax.numpy as jnp
from jax import lax
from jax.experimental import pallas as pl
from jax.experimental.pallas import tpu as pltpu
from jax.experimental.pallas import tpu_sc as plsc

_NUM_LEVELS = 16
_LEVEL_DIM = 2
_BASE_RES = 16
_MAX_RES = 2048
_LOG2_HASH = 19
_HASH = 2 ** _LOG2_HASH
_SCALE = math.exp((math.log(_MAX_RES) - math.log(_BASE_RES)) / (_NUM_LEVELS - 1))
_RES = [int(_BASE_RES * _SCALE ** l) for l in range(_NUM_LEVELS)]
_R2M = [(r * r) % _HASH for r in _RES]

_NC = 2
_NS = 16
_NW = _NC * _NS

_P = 512
_PB = _P // 128
_W = _NUM_LEVELS * _LEVEL_DIM
_FB = _W // 8
_REG = _P * 8


def _sc_body(xv, tabw, out, ca, cb, fa, fbv, ga, gb, s0, s1, *, n_points,
             rows_per_batch):
    ppw = n_points // _NW
    nchunk = ppw // _P
    wid = lax.axis_index("s") * _NC + lax.axis_index("c")
    wpb = rows_per_batch // ppw
    b = wid // wpb
    pb_base = (wid % wpb) * (ppw // 128)
    obase = b * (rows_per_batch * _W)

    def compute(k, cv, fv):
        pb0 = pb_base + k * _PB
        for c in range(3):
            pltpu.sync_copy(xv.at[c, pl.ds(pb0, _PB), b], cv.at[c])
        for pbl in range(_PB):
            def grp(jj, _, *, pbl=pbl):
                s = pl.ds(jj * 16, 16)
                v0 = cv[0, pbl, s]
                v1 = cv[1, pbl, s]
                v2 = cv[2, pbl, s]
                for l in range(_NUM_LEVELS):
                    res = _RES[l]
                    r2m = _R2M[l]
                    i0 = (v0 * res).astype(jnp.int32)
                    i1 = (v1 * res).astype(jnp.int32)
                    i2 = (v2 * res).astype(jnp.int32)
                    i0 = jnp.where(i0 >= res, i0 - res, i0)
                    i1 = jnp.where(i1 >= res, i1 - res, i1)
                    i2 = jnp.where(i2 >= res, i2 - res, i2)
                    h = i0 + i1 * res + i2 * r2m
                    fidx = (h & (_HASH - 1)) | (l << _LOG2_HASH)
                    lo = ((fidx >> 7) << 8) | (fidx & 127)
                    hi = lo | 128
                    off = ((l >> 2) * _REG + pbl * 1024
                           + (2 * (l & 3)) * 128)
                    fv[pl.ds(off + jj * 16, 16)] = lo
                    fv[pl.ds(off + 128 + jj * 16, 16)] = hi
                return 0

            lax.fori_loop(0, 8, grp, 0)

    def writeback(k, gv):
        pb0 = pb_base + k * _PB
        for f in range(_FB):
            pltpu.sync_copy(
                gv.at[pl.ds(f * _REG, _REG)],
                out.at[pl.ds(obase + f * (rows_per_batch * 8) + pb0 * 1024,
                             _REG)])

    def gwait(gv, sem):
        pltpu.make_async_copy(tabw.at[pl.ds(0, _P * _W)], gv, sem).wait()

    def iter_k(k, _):
        for par in (0, 1):
            cv, fv, gv, sem = (ca, fa, ga, s0) if par == 0 else (cb, fbv, gb, s1)
            pgv, psem = (gb, s1) if par == 0 else (ga, s0)

            @pl.when((k & 1) == par)
            def _(cv=cv, fv=fv, gv=gv, sem=sem, pgv=pgv, psem=psem):
                compute(k, cv, fv)
                pltpu.async_copy(tabw.at[fv], gv, sem)

                @pl.when(k > 0)
                def _():
                    gwait(pgv, psem)
                    writeback(k - 1, pgv)
        return 0

    lax.fori_loop(0, nchunk, iter_k, 0)
    last = nchunk - 1
    lgv, lsem = (ga, s0) if (last & 1) == 0 else (gb, s1)
    gwait(lgv, lsem)
    writeback(last, lgv)


def kernel(x, tables):
    b0, b1, _ = x.shape
    n = b0 * b1
    xv = x.reshape(b0, b1 // 128, 128, 3).transpose(3, 1, 0, 2)
    tabw = tables.reshape(_NUM_LEVELS, _HASH // 128, 128, _LEVEL_DIM)
    tabw = tabw.transpose(0, 1, 3, 2).reshape(-1)

    mesh = plsc.VectorSubcoreMesh(core_axis_name="c", subcore_axis_name="s")
    run = pl.kernel(
        functools.partial(_sc_body, n_points=n, rows_per_batch=b1),
        out_type=jax.ShapeDtypeStruct((n * _W,), jnp.float32),
        mesh=mesh,
        compiler_params=pltpu.CompilerParams(use_tc_tiling_on_sc=False),
        scratch_types=[
            pltpu.VMEM((3, _PB, 128), jnp.float32),
            pltpu.VMEM((3, _PB, 128), jnp.float32),
            pltpu.VMEM((_P * _W,), jnp.int32),
            pltpu.VMEM((_P * _W,), jnp.int32),
            pltpu.VMEM((_P * _W,), jnp.float32),
            pltpu.VMEM((_P * _W,), jnp.float32),
            pltpu.SemaphoreType.DMA,
            pltpu.SemaphoreType.DMA,
        ],
    )
    out = run(xv, tabw)
    out = out.reshape(b0, _FB, b1 // 128, 8, 128)
    return out.transpose(0, 2, 4, 1, 3).reshape(b0, b1, _W)


# speedup vs baseline: 31.7439x; 1.0308x over previous
import functools
import math

import jax
import jax.numpy as jnp
from jax import lax
from jax.experimental import pallas as pl
from jax.experimental.pallas import tpu as pltpu
from jax.experimental.pallas import tpu_sc as plsc

_NUM_LEVELS = 16
_LEVEL_DIM = 2
_BASE_RES = 16
_MAX_RES = 2048
_LOG2_HASH = 19
_HASH = 2 ** _LOG2_HASH
_SCALE = math.exp((math.log(_MAX_RES) - math.log(_BASE_RES)) / (_NUM_LEVELS - 1))
_RES = [int(_BASE_RES * _SCALE ** l) for l in range(_NUM_LEVELS)]
_R2M = [(r * r) % _HASH for r in _RES]

_NC = 2
_NS = 16
_NW = _NC * _NS

_P = 512
_PB = _P // 128
_W = _NUM_LEVELS * _LEVEL_DIM
_FB = _W // 8
_REG = _P * 8


def _sc_body(xv, tabw, out, ca, cb, fa0, fa1, fb0, fb1, ga0, ga1, gb0, gb1,
             sa0, sa1, sb0, sb1, *, n_points, rows_per_batch):
    ppw = n_points // _NW
    nchunk = ppw // _P
    wid = lax.axis_index("s") * _NC + lax.axis_index("c")
    wpb = rows_per_batch // ppw
    b = wid // wpb
    pb_base = (wid % wpb) * (ppw // 128)
    obase = b * (rows_per_batch * _W)
    half = _P * _W // 2

    def compute(k, cv, fv0, fv1):
        pb0 = pb_base + k * _PB
        for c in range(3):
            pltpu.sync_copy(xv.at[c, pl.ds(pb0, _PB), b], cv.at[c])
        for pbl in range(_PB):
            def grp(jj, _, *, pbl=pbl):
                s = pl.ds(jj * 16, 16)
                v0 = cv[0, pbl, s]
                v1 = cv[1, pbl, s]
                v2 = cv[2, pbl, s]
                for l in range(_NUM_LEVELS):
                    res = _RES[l]
                    r2m = _R2M[l]
                    i0 = (v0 * res).astype(jnp.int32)
                    i1 = (v1 * res).astype(jnp.int32)
                    i2 = (v2 * res).astype(jnp.int32)
                    i0 = jnp.where(i0 >= res, i0 - res, i0)
                    i1 = jnp.where(i1 >= res, i1 - res, i1)
                    i2 = jnp.where(i2 >= res, i2 - res, i2)
                    h = i0 + i1 * res + i2 * r2m
                    fidx = (h & (_HASH - 1)) | (l << _LOG2_HASH)
                    lo = ((fidx >> 7) << 8) | (fidx & 127)
                    hi = lo | 128
                    off = ((l >> 2) * _REG + pbl * 1024
                           + (2 * (l & 3)) * 128)
                    fv = fv0 if off < half else fv1
                    off = off % half
                    fv[pl.ds(off + jj * 16, 16)] = lo
                    fv[pl.ds(off + 128 + jj * 16, 16)] = hi
                return 0

            lax.fori_loop(0, 8, grp, 0)

    def writeback(k, gv0, gv1):
        pb0 = pb_base + k * _PB
        for f in range(_FB):
            gv = gv0 if f < _FB // 2 else gv1
            pltpu.sync_copy(
                gv.at[pl.ds((f % (_FB // 2)) * _REG, _REG)],
                out.at[pl.ds(obase + f * (rows_per_batch * 8) + pb0 * 1024,
                             _REG)])

    def gwait(gv, sem):
        pltpu.make_async_copy(tabw.at[pl.ds(0, half)], gv, sem).wait()

    bufs = (
        (ca, fa0, fa1, ga0, ga1, sa0, sa1),
        (cb, fb0, fb1, gb0, gb1, sb0, sb1),
    )

    def iter_k(k, _):
        for par in (0, 1):
            cv, fv0, fv1, gv0, gv1, se0, se1 = bufs[par]
            _, _, _, pg0, pg1, ps0, ps1 = bufs[1 - par]

            @pl.when((k & 1) == par)
            def _(cv=cv, fv0=fv0, fv1=fv1, gv0=gv0, gv1=gv1, se0=se0,
                  se1=se1, pg0=pg0, pg1=pg1, ps0=ps0, ps1=ps1):
                compute(k, cv, fv0, fv1)
                pltpu.async_copy(tabw.at[fv0], gv0, se0)
                pltpu.async_copy(tabw.at[fv1], gv1, se1)

                @pl.when(k > 0)
                def _():
                    gwait(pg0, ps0)
                    gwait(pg1, ps1)
                    writeback(k - 1, pg0, pg1)
        return 0

    lax.fori_loop(0, nchunk, iter_k, 0)
    last = nchunk - 1
    _, _, _, lg0, lg1, ls0, ls1 = bufs[last & 1]
    gwait(lg0, ls0)
    gwait(lg1, ls1)
    writeback(last, lg0, lg1)


def kernel(x, tables):
    b0, b1, _ = x.shape
    n = b0 * b1
    xv = x.reshape(b0, b1 // 128, 128, 3).transpose(3, 1, 0, 2)
    tabw = tables.reshape(_NUM_LEVELS, _HASH // 128, 128, _LEVEL_DIM)
    tabw = tabw.transpose(0, 1, 3, 2).reshape(-1)

    mesh = plsc.VectorSubcoreMesh(core_axis_name="c", subcore_axis_name="s")
    run = pl.kernel(
        functools.partial(_sc_body, n_points=n, rows_per_batch=b1),
        out_type=jax.ShapeDtypeStruct((n * _W,), jnp.float32),
        mesh=mesh,
        compiler_params=pltpu.CompilerParams(use_tc_tiling_on_sc=False),
        scratch_types=(
            [pltpu.VMEM((3, _PB, 128), jnp.float32)] * 2
            + [pltpu.VMEM((_P * _W // 2,), jnp.int32)] * 4
            + [pltpu.VMEM((_P * _W // 2,), jnp.float32)] * 4
            + [pltpu.SemaphoreType.DMA] * 4
        ),
    )
    out = run(xv, tabw)
    out = out.reshape(b0, _FB, b1 // 128, 8, 128)
    return out.transpose(0, 2, 4, 1, 3).reshape(b0, b1, _W)


# speedup vs baseline: 37.0394x; 1.1668x over previous
import functools
import math

import jax
import jax.numpy as jnp
from jax import lax
from jax.experimental import pallas as pl
from jax.experimental.pallas import tpu as pltpu
from jax.experimental.pallas import tpu_sc as plsc

_NUM_LEVELS = 16
_LEVEL_DIM = 2
_BASE_RES = 16
_MAX_RES = 2048
_LOG2_HASH = 19
_HASH = 2 ** _LOG2_HASH
_SCALE = math.exp((math.log(_MAX_RES) - math.log(_BASE_RES)) / (_NUM_LEVELS - 1))
_RES = [int(_BASE_RES * _SCALE ** l) for l in range(_NUM_LEVELS)]
_R2M = [(r * r) % _HASH for r in _RES]

_NC = 2
_NS = 16
_NW = _NC * _NS

_P = 512
_PB = _P // 128
_W = _NUM_LEVELS * _LEVEL_DIM
_FB = _W // 8
_REG = _P * 8
_N0 = _PB * 512 + _REG
_N1 = 2 * _REG
_TL0 = ((_RES[0] ** 3 - 1) >> 7 << 8) + 256
_TL1 = ((_RES[1] ** 3 - 1) >> 7 << 8) + 256


def _sc_body(xv, tabw, out, ca, cb, fa0, fa1, fb0, fb1, ga0, ga1, gb0, gb1,
             va, vb, tl0_v, tl1_v, sa0, sa1, sb0, sb1, *, n_points,
             rows_per_batch):
    ppw = n_points // _NW
    nchunk = ppw // _P
    wid = lax.axis_index("s") * _NC + lax.axis_index("c")
    wpb = rows_per_batch // ppw
    b = wid // wpb
    pb_base = (wid % wpb) * (ppw // 128)
    obase = b * (rows_per_batch * _W)

    pltpu.sync_copy(tabw.at[pl.ds(0, _TL0)], tl0_v)
    pltpu.sync_copy(tabw.at[pl.ds(1 << 20, _TL1)], tl1_v)

    def compute(k, cv, fv0, fv1, vv):
        pb0 = pb_base + k * _PB
        for c in range(3):
            pltpu.sync_copy(xv.at[c, pl.ds(pb0, _PB), b], cv.at[c])
        for pbl in range(_PB):
            def grp(jj, _, *, pbl=pbl):
                s = pl.ds(jj * 16, 16)
                v0 = cv[0, pbl, s]
                v1 = cv[1, pbl, s]
                v2 = cv[2, pbl, s]
                for l in range(_NUM_LEVELS):
                    res = _RES[l]
                    r2m = _R2M[l]
                    i0 = (v0 * res).astype(jnp.int32)
                    i1 = (v1 * res).astype(jnp.int32)
                    i2 = (v2 * res).astype(jnp.int32)
                    i0 = jnp.where(i0 >= res, i0 - res, i0)
                    i1 = jnp.where(i1 >= res, i1 - res, i1)
                    i2 = jnp.where(i2 >= res, i2 - res, i2)
                    h = i0 + i1 * res + i2 * r2m
                    if l < 2:
                        lo = ((h >> 7) << 8) | (h & 127)
                        hi = lo | 128
                        tl = tl0_v if l == 0 else tl1_v
                        g0 = plsc.load_gather(tl, [lo])
                        g1 = plsc.load_gather(tl, [hi])
                        voff = pbl * 512 + (2 * l) * 128 + jj * 16
                        vv[pl.ds(voff, 16)] = g0
                        vv[pl.ds(voff + 128, 16)] = g1
                        continue
                    fidx = (h & (_HASH - 1)) | (l << _LOG2_HASH)
                    lo = ((fidx >> 7) << 8) | (fidx & 127)
                    hi = lo | 128
                    if l < 4:
                        fv = fv0
                        off = pbl * 512 + (2 * (l - 2)) * 128
                    elif l < 8:
                        fv = fv0
                        off = _PB * 512 + pbl * 1024 + (2 * (l & 3)) * 128
                    else:
                        fv = fv1
                        off = (((l >> 2) - 2) * _REG + pbl * 1024
                               + (2 * (l & 3)) * 128)
                    fv[pl.ds(off + jj * 16, 16)] = lo
                    fv[pl.ds(off + 128 + jj * 16, 16)] = hi
                return 0

            lax.fori_loop(0, 8, grp, 0)

    def writeback(k, gv0, gv1, vv):
        pb0 = pb_base + k * _PB
        rpb8 = rows_per_batch * 8
        for pbl in range(_PB):
            dst = obase + (pb0 + pbl) * 1024
            pltpu.sync_copy(vv.at[pl.ds(pbl * 512, 512)],
                            out.at[pl.ds(dst, 512)])
            pltpu.sync_copy(gv0.at[pl.ds(pbl * 512, 512)],
                            out.at[pl.ds(dst + 512, 512)])
        pltpu.sync_copy(gv0.at[pl.ds(_PB * 512, _REG)],
                        out.at[pl.ds(obase + rpb8 + pb0 * 1024, _REG)])
        for f in (2, 3):
            pltpu.sync_copy(
                gv1.at[pl.ds((f - 2) * _REG, _REG)],
                out.at[pl.ds(obase + f * rpb8 + pb0 * 1024, _REG)])

    def gwait(gv, sem, nwords):
        pltpu.make_async_copy(tabw.at[pl.ds(0, nwords)], gv, sem).wait()

    bufs = (
        (ca, fa0, fa1, ga0, ga1, va, sa0, sa1),
        (cb, fb0, fb1, gb0, gb1, vb, sb0, sb1),
    )

    def iter_k(k, _):
        for par in (0, 1):
            cv, fv0, fv1, gv0, gv1, vv, se0, se1 = bufs[par]
            _, _, _, pg0, pg1, pvv, ps0, ps1 = bufs[1 - par]

            @pl.when((k & 1) == par)
            def _(cv=cv, fv0=fv0, fv1=fv1, gv0=gv0, gv1=gv1, vv=vv, se0=se0,
                  se1=se1, pg0=pg0, pg1=pg1, pvv=pvv, ps0=ps0, ps1=ps1):
                compute(k, cv, fv0, fv1, vv)
                pltpu.async_copy(tabw.at[fv0], gv0, se0)
                pltpu.async_copy(tabw.at[fv1], gv1, se1)

                @pl.when(k > 0)
                def _():
                    gwait(pg0, ps0, _N0)
                    gwait(pg1, ps1, _N1)
                    writeback(k - 1, pg0, pg1, pvv)
        return 0

    lax.fori_loop(0, nchunk, iter_k, 0)
    last = nchunk - 1
    _, _, _, lg0, lg1, lvv, ls0, ls1 = bufs[last & 1]
    gwait(lg0, ls0, _N0)
    gwait(lg1, ls1, _N1)
    writeback(last, lg0, lg1, lvv)


def kernel(x, tables):
    b0, b1, _ = x.shape
    n = b0 * b1
    xv = x.reshape(b0, b1 // 128, 128, 3).transpose(3, 1, 0, 2)
    tabw = tables.reshape(_NUM_LEVELS, _HASH // 128, 128, _LEVEL_DIM)
    tabw = tabw.transpose(0, 1, 3, 2).reshape(-1)

    mesh = plsc.VectorSubcoreMesh(core_axis_name="c", subcore_axis_name="s")
    run = pl.kernel(
        functools.partial(_sc_body, n_points=n, rows_per_batch=b1),
        out_type=jax.ShapeDtypeStruct((n * _W,), jnp.float32),
        mesh=mesh,
        compiler_params=pltpu.CompilerParams(use_tc_tiling_on_sc=False, needs_layout_passes=False),
        scratch_types=(
            [pltpu.VMEM((3, _PB, 128), jnp.float32)] * 2
            + [pltpu.VMEM((_N0,), jnp.int32), pltpu.VMEM((_N1,), jnp.int32)] * 2
            + [pltpu.VMEM((_N0,), jnp.float32), pltpu.VMEM((_N1,), jnp.float32)] * 2
            + [pltpu.VMEM((_P * 4,), jnp.float32)] * 2
            + [pltpu.VMEM((_TL0,), jnp.float32), pltpu.VMEM((_TL1,), jnp.float32)]
            + [pltpu.SemaphoreType.DMA] * 4
        ),
    )
    out = run(xv, tabw)
    out = out.reshape(b0, _FB, b1 // 128, 8, 128)
    return out.transpose(0, 2, 4, 1, 3).reshape(b0, b1, _W)


# speedup vs baseline: 39.9770x; 1.0793x over previous
import functools
import math

import jax
import jax.numpy as jnp
from jax import lax
from jax.experimental import pallas as pl
from jax.experimental.pallas import tpu as pltpu
from jax.experimental.pallas import tpu_sc as plsc

_NUM_LEVELS = 16
_LEVEL_DIM = 2
_BASE_RES = 16
_MAX_RES = 2048
_LOG2_HASH = 19
_HASH = 2 ** _LOG2_HASH
_SCALE = math.exp((math.log(_MAX_RES) - math.log(_BASE_RES)) / (_NUM_LEVELS - 1))
_RES = [int(_BASE_RES * _SCALE ** l) for l in range(_NUM_LEVELS)]
_R2M = [(r * r) % _HASH for r in _RES]

_NC = 2
_NS = 16
_NW = _NC * _NS

_P = 256
_PB = _P // 128
_W = _NUM_LEVELS * _LEVEL_DIM
_FB = _W // 8
_REG = _P * 8
_N0 = _PB * 256 + _REG
_N1 = 2 * _REG
_TL0 = ((_RES[0] ** 3 - 1) >> 7 << 8) + 256
_TL1 = ((_RES[1] ** 3 - 1) >> 7 << 8) + 256
_TL2 = ((_RES[2] ** 3 - 1) >> 7 << 8) + 256
_VV = _PB * 768


def _sc_body(xv, tabw, out, ca, cb, fa0, fa1, fb0, fb1, ga0, ga1, gb0, gb1,
             va, vb, tl0_v, tl1_v, tl2_v, sa0, sa1, sb0, sb1, *, n_points,
             rows_per_batch):
    ppw = n_points // _NW
    nchunk = ppw // _P
    wid = lax.axis_index("s") * _NC + lax.axis_index("c")
    wpb = rows_per_batch // ppw
    b = wid // wpb
    pb_base = (wid % wpb) * (ppw // 128)
    obase = b * (rows_per_batch * _W)

    pltpu.sync_copy(tabw.at[pl.ds(0, _TL0)], tl0_v)
    pltpu.sync_copy(tabw.at[pl.ds(1 << 20, _TL1)], tl1_v)
    pltpu.sync_copy(tabw.at[pl.ds(2 << 20, _TL2)], tl2_v)

    def compute(k, cv, fv0, fv1, vv):
        pb0 = pb_base + k * _PB
        for c in range(3):
            pltpu.sync_copy(xv.at[c, pl.ds(pb0, _PB), b], cv.at[c])
        for pbl in range(_PB):
            def grp(jj, _, *, pbl=pbl):
                s = pl.ds(jj * 16, 16)
                v0 = cv[0, pbl, s]
                v1 = cv[1, pbl, s]
                v2 = cv[2, pbl, s]
                for l in range(_NUM_LEVELS):
                    res = _RES[l]
                    r2m = _R2M[l]
                    i0 = (v0 * res).astype(jnp.int32)
                    i1 = (v1 * res).astype(jnp.int32)
                    i2 = (v2 * res).astype(jnp.int32)
                    i0 = jnp.where(i0 >= res, i0 - res, i0)
                    i1 = jnp.where(i1 >= res, i1 - res, i1)
                    i2 = jnp.where(i2 >= res, i2 - res, i2)
                    h = i0 + i1 * res + i2 * r2m
                    if l < 3:
                        lo = ((h >> 7) << 8) | (h & 127)
                        hi = lo | 128
                        tl = (tl0_v, tl1_v, tl2_v)[l]
                        g0 = plsc.load_gather(tl, [lo])
                        g1 = plsc.load_gather(tl, [hi])
                        voff = pbl * 768 + (2 * l) * 128 + jj * 16
                        vv[pl.ds(voff, 16)] = g0
                        vv[pl.ds(voff + 128, 16)] = g1
                        continue
                    fidx = (h & (_HASH - 1)) | (l << _LOG2_HASH)
                    lo = ((fidx >> 7) << 8) | (fidx & 127)
                    hi = lo | 128
                    if l < 4:
                        fv = fv0
                        off = pbl * 256
                    elif l < 8:
                        fv = fv0
                        off = _PB * 256 + pbl * 1024 + (2 * (l & 3)) * 128
                    else:
                        fv = fv1
                        off = (((l >> 2) - 2) * _REG + pbl * 1024
                               + (2 * (l & 3)) * 128)
                    fv[pl.ds(off + jj * 16, 16)] = lo
                    fv[pl.ds(off + 128 + jj * 16, 16)] = hi
                return 0

            lax.fori_loop(0, 8, grp, 0)

    def writeback(k, gv0, gv1, vv):
        pb0 = pb_base + k * _PB
        rpb8 = rows_per_batch * 8
        for pbl in range(_PB):
            dst = obase + (pb0 + pbl) * 1024
            pltpu.sync_copy(vv.at[pl.ds(pbl * 768, 768)],
                            out.at[pl.ds(dst, 768)])
            pltpu.sync_copy(gv0.at[pl.ds(pbl * 256, 256)],
                            out.at[pl.ds(dst + 768, 256)])
        pltpu.sync_copy(gv0.at[pl.ds(_PB * 256, _REG)],
                        out.at[pl.ds(obase + rpb8 + pb0 * 1024, _REG)])
        for f in (2, 3):
            pltpu.sync_copy(
                gv1.at[pl.ds((f - 2) * _REG, _REG)],
                out.at[pl.ds(obase + f * rpb8 + pb0 * 1024, _REG)])

    def gwait(gv, sem, nwords):
        pltpu.make_async_copy(tabw.at[pl.ds(0, nwords)], gv, sem).wait()

    bufs = (
        (ca, fa0, fa1, ga0, ga1, va, sa0, sa1),
        (cb, fb0, fb1, gb0, gb1, vb, sb0, sb1),
    )

    def iter_k(k, _):
        for par in (0, 1):
            cv, fv0, fv1, gv0, gv1, vv, se0, se1 = bufs[par]
            _, _, _, pg0, pg1, pvv, ps0, ps1 = bufs[1 - par]

            @pl.when((k & 1) == par)
            def _(cv=cv, fv0=fv0, fv1=fv1, gv0=gv0, gv1=gv1, vv=vv, se0=se0,
                  se1=se1, pg0=pg0, pg1=pg1, pvv=pvv, ps0=ps0, ps1=ps1):
                compute(k, cv, fv0, fv1, vv)
                pltpu.async_copy(tabw.at[fv0], gv0, se0)
                pltpu.async_copy(tabw.at[fv1], gv1, se1)

                @pl.when(k > 0)
                def _():
                    gwait(pg0, ps0, _N0)
                    gwait(pg1, ps1, _N1)
                    writeback(k - 1, pg0, pg1, pvv)
        return 0

    lax.fori_loop(0, nchunk, iter_k, 0)
    last = nchunk - 1
    _, _, _, lg0, lg1, lvv, ls0, ls1 = bufs[last & 1]
    gwait(lg0, ls0, _N0)
    gwait(lg1, ls1, _N1)
    writeback(last, lg0, lg1, lvv)


def kernel(x, tables):
    b0, b1, _ = x.shape
    n = b0 * b1
    xv = x.reshape(b0, b1 // 128, 128, 3).transpose(3, 1, 0, 2)
    tabw = tables.reshape(_NUM_LEVELS, _HASH // 128, 128, _LEVEL_DIM)
    tabw = tabw.transpose(0, 1, 3, 2).reshape(-1)

    mesh = plsc.VectorSubcoreMesh(core_axis_name="c", subcore_axis_name="s")
    run = pl.kernel(
        functools.partial(_sc_body, n_points=n, rows_per_batch=b1),
        out_type=jax.ShapeDtypeStruct((n * _W,), jnp.float32),
        mesh=mesh,
        compiler_params=pltpu.CompilerParams(use_tc_tiling_on_sc=False, needs_layout_passes=False),
        scratch_types=(
            [pltpu.VMEM((3, _PB, 128), jnp.float32)] * 2
            + [pltpu.VMEM((_N0,), jnp.int32), pltpu.VMEM((_N1,), jnp.int32)] * 2
            + [pltpu.VMEM((_N0,), jnp.float32), pltpu.VMEM((_N1,), jnp.float32)] * 2
            + [pltpu.VMEM((_VV,), jnp.float32)] * 2
            + [pltpu.VMEM((_TL0,), jnp.float32), pltpu.VMEM((_TL1,), jnp.float32),
               pltpu.VMEM((_TL2,), jnp.float32)]
            + [pltpu.SemaphoreType.DMA] * 4
        ),
    )
    out = run(xv, tabw)
    out = out.reshape(b0, _FB, b1 // 128, 8, 128)
    return out.transpose(0, 2, 4, 1, 3).reshape(b0, b1, _W)
